# Initial kernel scaffold; baseline (speedup 1.0000x reference)
import jax
import jax.numpy as jnp
from jax.experimental import pallas as pl


def kernel(x, g_id, embedding):
    raise NotImplementedError("write your pallas kernel here")


# speedup vs baseline: 1.0352x; 1.0352x over previous
import jax
import jax.numpy as jnp
from jax.experimental import pallas as pl

B_BLK = 32


def _body(g_ref, x_ref, emb_ref, o_ref):
    g = g_ref[0, 0, :]
    nunq = emb_ref.shape[0]
    oh = (g[:, None] == jax.lax.broadcasted_iota(jnp.int32, (B_BLK, nunq), 1))
    emb = jnp.dot(oh.astype(jnp.float32), emb_ref[...],
                  preferred_element_type=jnp.float32)
    o_ref[...] = x_ref[...] + emb[:, None, :]


def kernel(x, g_id, embedding):
    B, L, H = x.shape
    nb = B // B_BLK
    g3 = g_id.astype(jnp.int32).reshape(nb, 1, B_BLK)
    return pl.pallas_call(
        _body,
        grid=(nb,),
        in_specs=[
            pl.BlockSpec((1, 1, B_BLK), lambda i: (i, 0, 0)),
            pl.BlockSpec((B_BLK, L, H), lambda i: (i, 0, 0)),
            pl.BlockSpec(embedding.shape, lambda i: (0, 0)),
        ],
        out_specs=pl.BlockSpec((B_BLK, L, H), lambda i: (i, 0, 0)),
        out_shape=jax.ShapeDtypeStruct((B, L, H), x.dtype),
    )(g3, x, embedding)


# speedup vs baseline: 1.0598x; 1.0237x over previous
import jax
import jax.numpy as jnp
from jax.experimental import pallas as pl

B_BLK = 64


def _body(g_ref, x_ref, emb_ref, o_ref):
    g = g_ref[0, 0, :]
    nunq = emb_ref.shape[0]
    oh = (g[:, None] == jax.lax.broadcasted_iota(jnp.int32, (B_BLK, nunq), 1))
    emb = jnp.dot(oh.astype(jnp.float32), emb_ref[...],
                  preferred_element_type=jnp.float32)
    o_ref[...] = x_ref[...] + emb[:, None, :]


def kernel(x, g_id, embedding):
    B, L, H = x.shape
    nb = B // B_BLK
    g3 = g_id.astype(jnp.int32).reshape(nb, 1, B_BLK)
    return pl.pallas_call(
        _body,
        grid=(nb,),
        in_specs=[
            pl.BlockSpec((1, 1, B_BLK), lambda i: (i, 0, 0)),
            pl.BlockSpec((B_BLK, L, H), lambda i: (i, 0, 0)),
            pl.BlockSpec(embedding.shape, lambda i: (0, 0)),
        ],
        out_specs=pl.BlockSpec((B_BLK, L, H), lambda i: (i, 0, 0)),
        out_shape=jax.ShapeDtypeStruct((B, L, H), x.dtype),
    )(g3, x, embedding)


# speedup vs baseline: 1.0659x; 1.0058x over previous
import jax
import jax.numpy as jnp
from jax.experimental import pallas as pl

B_BLK = 128


def _body(g_ref, x_ref, emb_ref, o_ref):
    g = g_ref[0, 0, :]
    nunq = emb_ref.shape[0]
    oh = (g[:, None] == jax.lax.broadcasted_iota(jnp.int32, (B_BLK, nunq), 1))
    emb = jnp.dot(oh.astype(jnp.float32), emb_ref[...],
                  preferred_element_type=jnp.float32)
    o_ref[...] = x_ref[...] + emb[:, None, :]


def kernel(x, g_id, embedding):
    B, L, H = x.shape
    nb = B // B_BLK
    g3 = g_id.astype(jnp.int32).reshape(nb, 1, B_BLK)
    return pl.pallas_call(
        _body,
        grid=(nb,),
        in_specs=[
            pl.BlockSpec((1, 1, B_BLK), lambda i: (i, 0, 0)),
            pl.BlockSpec((B_BLK, L, H), lambda i: (i, 0, 0)),
            pl.BlockSpec(embedding.shape, lambda i: (0, 0)),
        ],
        out_specs=pl.BlockSpec((B_BLK, L, H), lambda i: (i, 0, 0)),
        out_shape=jax.ShapeDtypeStruct((B, L, H), x.dtype),
    )(g3, x, embedding)
